# baseline (speedup 1.0000x reference)
import jax
import jax.numpy as jnp
from jax.experimental import pallas as pl

_SIM_THRESHOLD = 0.9
_EPS = 1e-08
_NB = 8


def _merge_kernel(slots_ref, out_ref, mask_out_ref):
    nb, S, D = slots_ref.shape
    iota_col = jax.lax.broadcasted_iota(jnp.int32, (S, S), 1)
    iota_row = jax.lax.broadcasted_iota(jnp.int32, (S, S), 0)
    mask_rows = []
    for b in range(nb):
        x = slots_ref[b]
        sq = jnp.sum(x * x, axis=-1, keepdims=True)
        norm = jnp.sqrt(sq)
        gram = jax.lax.dot_general(
            x, x, (((1,), (1,)), ((), ())), preferred_element_type=jnp.float32
        )
        sim = gram / (norm * norm.T + _EPS)
        mask = sim > _SIM_THRESHOLD
        maskf = mask.astype(jnp.float32)
        c = jnp.sum(maskf, axis=-1, keepdims=True)
        multi = c > 1.0
        single = c <= 1.0
        min_idx = jnp.min(jnp.where(mask, iota_col, S), axis=-1, keepdims=True)
        zero_hit = mask & multi & (iota_col != min_idx)
        zeroed = jnp.any(zero_hit, axis=0, keepdims=True)
        alive = (c.T > 0.0) & jnp.logical_not(zeroed)
        mask_rows.append(alive.astype(jnp.float32))
        candidate = multi & (min_idx == iota_col)
        writer = jnp.max(
            jnp.where(candidate, iota_row, -1), axis=0, keepdims=True
        )
        p_sel = ((writer.T == iota_col) & multi).astype(jnp.float32)
        n_rows = maskf / (c + _EPS)
        m_diag = jnp.where((iota_row == iota_col) & single, 1.0, 0.0)
        merge_m = m_diag + jax.lax.dot_general(
            p_sel, n_rows, (((1,), (0,)), ((), ())),
            preferred_element_type=jnp.float32,
        )
        out_ref[b] = jax.lax.dot_general(
            merge_m, x, (((1,), (0,)), ((), ())),
            preferred_element_type=jnp.float32,
        )
    mask_out_ref[...] = jnp.concatenate(mask_rows, axis=0)


def kernel(slots):
    B, S, D = slots.shape
    grid = (B // _NB,)
    final_slots, slot_mask = pl.pallas_call(
        _merge_kernel,
        grid=grid,
        in_specs=[pl.BlockSpec((_NB, S, D), lambda i: (i, 0, 0))],
        out_specs=[
            pl.BlockSpec((_NB, S, D), lambda i: (i, 0, 0)),
            pl.BlockSpec((_NB, S), lambda i: (i, 0)),
        ],
        out_shape=[
            jax.ShapeDtypeStruct((B, S, D), jnp.float32),
            jax.ShapeDtypeStruct((B, S), jnp.float32),
        ],
    )(slots)
    return final_slots, slot_mask


# speedup vs baseline: 2.9881x; 2.9881x over previous
import jax
import jax.numpy as jnp
from jax import lax
from jax.experimental import pallas as pl

_SIM_THRESHOLD = 0.9
_EPS = 1e-08
_NB = 8

_BATCH_DOT = (((2,), (2,)), ((0,), (0,)))
_BATCH_MM = (((2,), (1,)), ((0,), (0,)))


def _merge_kernel(slots_ref, out_ref, mask_out_ref):
    nb, S, D = slots_ref.shape
    x = slots_ref[...]
    sq = jnp.sum(x * x, axis=-1, keepdims=True)
    norm = jnp.sqrt(sq)
    gram = lax.dot_general(x, x, _BATCH_DOT, preferred_element_type=jnp.float32)
    thresh = _SIM_THRESHOLD * (norm * jnp.swapaxes(norm, 1, 2) + _EPS)
    mask = gram > thresh
    maskf = mask.astype(jnp.float32)
    c = jnp.sum(maskf, axis=-1, keepdims=True)
    multi = c > 1.0
    single = c <= 1.0
    iota_col = lax.broadcasted_iota(jnp.int32, (nb, S, S), 2)
    iota_row = lax.broadcasted_iota(jnp.int32, (nb, S, S), 1)
    min_idx = jnp.min(jnp.where(mask, iota_col, S), axis=-1, keepdims=True)
    zero_hit = mask & multi & (iota_col != min_idx)
    zeroed = jnp.any(zero_hit, axis=1, keepdims=True)
    alive = (jnp.swapaxes(c, 1, 2) > 0.0) & jnp.logical_not(zeroed)
    mask_out_ref[...] = alive.astype(jnp.float32).reshape(nb, S)
    candidate = multi & (min_idx == iota_col)
    writer = jnp.max(jnp.where(candidate, iota_row, -1), axis=1, keepdims=True)
    p_sel = ((jnp.swapaxes(writer, 1, 2) == iota_col) & multi).astype(jnp.float32)
    n_rows = maskf / (c + _EPS)
    m_diag = jnp.where((iota_row == iota_col) & single, 1.0, 0.0)
    merge_m = m_diag + lax.dot_general(
        p_sel, n_rows, _BATCH_MM, preferred_element_type=jnp.float32
    )
    out_ref[...] = lax.dot_general(
        merge_m, x, _BATCH_MM, preferred_element_type=jnp.float32
    )


def kernel(slots):
    B, S, D = slots.shape
    grid = (B // _NB,)
    final_slots, slot_mask = pl.pallas_call(
        _merge_kernel,
        grid=grid,
        in_specs=[pl.BlockSpec((_NB, S, D), lambda i: (i, 0, 0))],
        out_specs=[
            pl.BlockSpec((_NB, S, D), lambda i: (i, 0, 0)),
            pl.BlockSpec((_NB, S), lambda i: (i, 0)),
        ],
        out_shape=[
            jax.ShapeDtypeStruct((B, S, D), jnp.float32),
            jax.ShapeDtypeStruct((B, S), jnp.float32),
        ],
    )(slots)
    return final_slots, slot_mask


# speedup vs baseline: 3.9495x; 1.3217x over previous
import jax
import jax.numpy as jnp
from jax import lax
from jax.experimental import pallas as pl

_SIM_THRESHOLD = 0.9
_EPS = 1e-08
_NB = 8

_BATCH_DOT = (((2,), (2,)), ((0,), (0,)))
_BATCH_MM = (((2,), (1,)), ((0,), (0,)))


def _merge_kernel(slots_ref, out_ref, mask_out_ref):
    nb, S, D = slots_ref.shape
    x = slots_ref[...]
    sq = jnp.sum(x * x, axis=-1, keepdims=True)
    norm = jnp.sqrt(sq)
    gram = lax.dot_general(x, x, _BATCH_DOT, preferred_element_type=jnp.float32)
    thresh = _SIM_THRESHOLD * (norm * jnp.swapaxes(norm, 1, 2) + _EPS)
    mask = gram > thresh
    maskf = mask.astype(jnp.float32)
    c = jnp.sum(maskf, axis=-1, keepdims=True)
    multi = c > 1.0
    nonempty = jnp.swapaxes(c, 1, 2) > 0.0
    any_multi = jnp.any(multi)

    @pl.when(jnp.logical_not(any_multi))
    def _fast():
        out_ref[...] = x
        mask_out_ref[...] = nonempty.astype(jnp.float32).reshape(nb, S)

    @pl.when(any_multi)
    def _general():
        single = jnp.logical_not(multi)
        iota_col = lax.broadcasted_iota(jnp.int32, (nb, S, S), 2)
        iota_row = lax.broadcasted_iota(jnp.int32, (nb, S, S), 1)
        min_idx = jnp.min(jnp.where(mask, iota_col, S), axis=-1, keepdims=True)
        zero_hit = mask & multi & (iota_col != min_idx)
        zeroed = jnp.any(zero_hit, axis=1, keepdims=True)
        alive = nonempty & jnp.logical_not(zeroed)
        mask_out_ref[...] = alive.astype(jnp.float32).reshape(nb, S)
        candidate = multi & (min_idx == iota_col)
        writer = jnp.max(
            jnp.where(candidate, iota_row, -1), axis=1, keepdims=True
        )
        p_sel = ((jnp.swapaxes(writer, 1, 2) == iota_col) & multi).astype(
            jnp.float32
        )
        n_rows = maskf / (c + _EPS)
        m_diag = jnp.where((iota_row == iota_col) & single, 1.0, 0.0)
        merge_m = m_diag + lax.dot_general(
            p_sel, n_rows, _BATCH_MM, preferred_element_type=jnp.float32
        )
        out_ref[...] = lax.dot_general(
            merge_m, x, _BATCH_MM, preferred_element_type=jnp.float32
        )


def kernel(slots):
    B, S, D = slots.shape
    grid = (B // _NB,)
    final_slots, slot_mask = pl.pallas_call(
        _merge_kernel,
        grid=grid,
        in_specs=[pl.BlockSpec((_NB, S, D), lambda i: (i, 0, 0))],
        out_specs=[
            pl.BlockSpec((_NB, S, D), lambda i: (i, 0, 0)),
            pl.BlockSpec((_NB, S), lambda i: (i, 0)),
        ],
        out_shape=[
            jax.ShapeDtypeStruct((B, S, D), jnp.float32),
            jax.ShapeDtypeStruct((B, S), jnp.float32),
        ],
    )(slots)
    return final_slots, slot_mask
